# Initial kernel scaffold; baseline (speedup 1.0000x reference)
import jax
import jax.numpy as jnp
from jax.experimental import pallas as pl


def kernel(inputs, emb_table, pos_table):
    raise NotImplementedError("write your pallas kernel here")


# speedup vs baseline: 1.2358x; 1.2358x over previous
import functools

import jax
import jax.numpy as jnp
from jax import lax
from jax.experimental import pallas as pl
from jax.experimental.pallas import tpu as pltpu
from jax.experimental.pallas import tpu_sc as plsc

_LANES = 16


@functools.lru_cache(maxsize=None)
def _build(batch, seq, vocab, d_model):
    info = plsc.get_sparse_core_info()
    nc, ns = info.num_cores, info.num_subcores
    nw = nc * ns
    tok = batch * seq
    tpw = tok // nw
    chunk = 32
    nchunk = tpw // chunk
    vecs = d_model // _LANES

    assert tok % nw == 0 and tpw % chunk == 0 and d_model % _LANES == 0
    assert seq % tpw == 0

    mesh = plsc.VectorSubcoreMesh(core_axis_name="c", subcore_axis_name="s")

    @functools.partial(
        pl.kernel,
        mesh=mesh,
        out_type=jax.ShapeDtypeStruct((tok, d_model), jnp.float32),
        scratch_types=[
            pltpu.VMEM((tpw,), jnp.int32),
            pltpu.VMEM((2, chunk, d_model), jnp.float32),
            pltpu.VMEM((2, chunk, d_model), jnp.float32),
            pltpu.SemaphoreType.DMA,
            pltpu.SemaphoreType.DMA,
            pltpu.SemaphoreType.DMA,
        ],
    )
    def k(ids_hbm, emb_hbm, pos_hbm, out_hbm, idx_v, emb_v, pos_v,
          sem_g, sem_p, sem_o):
        wid = lax.axis_index("s") * nc + lax.axis_index("c")
        base = wid * tpw
        seq_base = lax.rem(base, seq)

        pltpu.sync_copy(ids_hbm.at[pl.ds(base, tpw)], idx_v)

        def start(c):
            buf = c % 2
            g = pltpu.async_copy(
                emb_hbm.at[idx_v.at[pl.ds(c * chunk, chunk)]],
                emb_v.at[buf], sem_g)
            p = pltpu.async_copy(
                pos_hbm.at[pl.ds(seq_base + c * chunk, chunk)],
                pos_v.at[buf], sem_p)
            return g, p

        def add_rows(c):
            buf = c % 2
            ec = emb_v.at[buf]
            pc = pos_v.at[buf]

            def body(t, _):
                for j in range(vecs):
                    sl = pl.ds(j * _LANES, _LANES)
                    plsc.addupdate(ec.at[t, sl], pc[t, sl])
                return 0

            lax.fori_loop(0, chunk, body, 0)

        pending = {0: start(0)}
        stores = {}
        for c in range(nchunk):
            if c + 1 < nchunk:
                if c - 1 in stores:
                    stores.pop(c - 1).wait()
                pending[c + 1] = start(c + 1)
            g, p = pending.pop(c)
            g.wait()
            p.wait()
            add_rows(c)
            stores[c] = pltpu.async_copy(
                emb_v.at[c % 2],
                out_hbm.at[pl.ds(base + c * chunk, chunk)], sem_o)
        for c in sorted(stores):
            stores[c].wait()

    return k


def kernel(inputs, emb_table, pos_table):
    batch, seq = inputs.shape
    vocab, d_model = emb_table.shape
    ids = inputs.reshape(-1).astype(jnp.int32)
    out = _build(batch, seq, vocab, d_model)(ids, emb_table, pos_table)
    return out.reshape(batch, seq, d_model)


# speedup vs baseline: 1.2595x; 1.0192x over previous
import functools

import jax
import jax.numpy as jnp
from jax import lax
from jax.experimental import pallas as pl
from jax.experimental.pallas import tpu as pltpu
from jax.experimental.pallas import tpu_sc as plsc

_LANES = 16


@functools.lru_cache(maxsize=None)
def _build(batch, seq, vocab, d_model):
    info = plsc.get_sparse_core_info()
    nc, ns = info.num_cores, info.num_subcores
    nw = nc * ns
    spw = seq // nw
    cs = 16
    nchunk = spw // cs
    rows = batch * cs
    vecs = d_model // _LANES

    assert seq % nw == 0 and spw % cs == 0 and d_model % _LANES == 0

    mesh = plsc.VectorSubcoreMesh(core_axis_name="c", subcore_axis_name="s")

    @functools.partial(
        pl.kernel,
        mesh=mesh,
        out_type=jax.ShapeDtypeStruct((batch * seq, d_model), jnp.float32),
        scratch_types=[
            pltpu.VMEM((nchunk, rows), jnp.int32),
            pltpu.VMEM((2, rows, d_model), jnp.float32),
            pltpu.VMEM((2, cs, d_model), jnp.float32),
            pltpu.SemaphoreType.DMA,
            pltpu.SemaphoreType.DMA,
            pltpu.SemaphoreType.DMA,
        ],
    )
    def k(ids_hbm, emb_hbm, pos_hbm, out_hbm, idx_v, emb_v, pos_v,
          sem_g, sem_p, sem_o):
        wid = lax.axis_index("s") * nc + lax.axis_index("c")
        s0 = wid * spw

        for c in range(nchunk):
            for b in range(batch):
                pltpu.sync_copy(
                    ids_hbm.at[pl.ds(b * seq + s0 + c * cs, cs)],
                    idx_v.at[c, pl.ds(b * cs, cs)])

        def start(c):
            buf = c % 2
            g = pltpu.async_copy(emb_hbm.at[idx_v.at[c]], emb_v.at[buf],
                                 sem_g)
            p = pltpu.async_copy(pos_hbm.at[pl.ds(s0 + c * cs, cs)],
                                 pos_v.at[buf], sem_p)
            return g, p

        def add_rows(c):
            buf = c % 2
            ec = emb_v.at[buf]
            pc = pos_v.at[buf]

            def body(t, _):
                for j in range(vecs):
                    sl = pl.ds(j * _LANES, _LANES)
                    pv = pc[t, sl]
                    for b in range(batch):
                        plsc.addupdate(ec.at[b * cs + t, sl], pv)
                return 0

            lax.fori_loop(0, cs, body, 0)

        def store(c):
            buf = c % 2
            return [
                pltpu.async_copy(
                    emb_v.at[buf, pl.ds(b * cs, cs)],
                    out_hbm.at[pl.ds(b * seq + s0 + c * cs, cs)], sem_o)
                for b in range(batch)
            ]

        pending = {0: start(0)}
        stores = {}
        for c in range(nchunk):
            if c + 1 < nchunk:
                if c - 1 in stores:
                    for h in stores.pop(c - 1):
                        h.wait()
                pending[c + 1] = start(c + 1)
            g, p = pending.pop(c)
            g.wait()
            p.wait()
            add_rows(c)
            stores[c] = store(c)
        for c in sorted(stores):
            for h in stores[c]:
                h.wait()

    return k


def kernel(inputs, emb_table, pos_table):
    batch, seq = inputs.shape
    vocab, d_model = emb_table.shape
    ids = inputs.reshape(-1).astype(jnp.int32)
    out = _build(batch, seq, vocab, d_model)(ids, emb_table, pos_table)
    return out.reshape(batch, seq, d_model)


# speedup vs baseline: 1.4324x; 1.1373x over previous
import functools

import jax
import jax.numpy as jnp
from jax import lax
from jax.experimental import pallas as pl
from jax.experimental.pallas import tpu as pltpu
from jax.experimental.pallas import tpu_sc as plsc

_LANES = 16


@functools.lru_cache(maxsize=None)
def _build(batch, seq, vocab, d_model):
    info = plsc.get_sparse_core_info()
    nc, ns = info.num_cores, info.num_subcores
    nw = nc * ns
    spw = seq // nw
    cs = 16
    nchunk = spw // cs
    vecs = d_model // _LANES

    assert seq % nw == 0 and spw % cs == 0 and d_model % _LANES == 0

    mesh = plsc.VectorSubcoreMesh(core_axis_name="c", subcore_axis_name="s")

    @functools.partial(
        pl.kernel,
        mesh=mesh,
        out_type=jax.ShapeDtypeStruct((batch, seq, d_model), jnp.float32),
        scratch_types=[
            pltpu.VMEM((batch, spw), jnp.int32),
            pltpu.VMEM((2, batch * cs, d_model), jnp.float32),
            pltpu.VMEM((2, cs, d_model), jnp.float32),
            pltpu.SemaphoreType.DMA,
            pltpu.SemaphoreType.DMA,
            pltpu.SemaphoreType.DMA,
        ],
    )
    def k(ids_hbm, emb_hbm, pos_hbm, out_hbm, idx_v, emb_v, pos_v,
          sem_g, sem_p, sem_o):
        wid = lax.axis_index("s") * nc + lax.axis_index("c")
        s0 = wid * spw

        idx_cps = [
            pltpu.async_copy(ids_hbm.at[b, pl.ds(s0, spw)], idx_v.at[b],
                             sem_g)
            for b in range(batch)
        ]
        for h in idx_cps:
            h.wait()

        def start(c):
            buf = c % 2
            g = [
                pltpu.async_copy(
                    emb_hbm.at[idx_v.at[b, pl.ds(c * cs, cs)]],
                    emb_v.at[buf, pl.ds(b * cs, cs)], sem_g)
                for b in range(batch)
            ]
            p = pltpu.async_copy(pos_hbm.at[pl.ds(s0 + c * cs, cs)],
                                 pos_v.at[buf], sem_p)
            return g + [p]

        def add_rows(c):
            buf = c % 2
            ec = emb_v.at[buf]
            pc = pos_v.at[buf]

            def body(t, _):
                for j in range(vecs):
                    sl = pl.ds(j * _LANES, _LANES)
                    pv = pc[t, sl]
                    for b in range(batch):
                        plsc.addupdate(ec.at[b * cs + t, sl], pv)
                return 0

            lax.fori_loop(0, cs, body, 0)

        def store(c):
            buf = c % 2
            return [
                pltpu.async_copy(
                    emb_v.at[buf, pl.ds(b * cs, cs)],
                    out_hbm.at[b, pl.ds(s0 + c * cs, cs)], sem_o)
                for b in range(batch)
            ]

        pending = {0: start(0)}
        stores = {}
        for c in range(nchunk):
            if c + 1 < nchunk:
                if c - 1 in stores:
                    for h in stores.pop(c - 1):
                        h.wait()
                pending[c + 1] = start(c + 1)
            for h in pending.pop(c):
                h.wait()
            add_rows(c)
            stores[c] = store(c)
        for c in sorted(stores):
            for h in stores[c]:
                h.wait()

    return k


def kernel(inputs, emb_table, pos_table):
    batch, seq = inputs.shape
    vocab, d_model = emb_table.shape
    return _build(batch, seq, vocab, d_model)(inputs, emb_table, pos_table)
